# baseline (speedup 1.0000x reference)
import functools

import jax
import jax.numpy as jnp
from jax import lax
from jax.experimental import pallas as pl
from jax.experimental.pallas import tpu as pltpu
from jax.experimental.pallas import tpu_sc as plsc

B, H, W, C = 4, 192, 192, 96
OH, OW = 2 * H, 2 * W
N_ROWS = B * H * W
HW = H * W
PLANE_B = OH * OW
OUT_WORDS = PLANE_B * C * B

NUM_SC = 2
NUM_TILES = 16
CH_PER_SC = 8
CH_PER_PASS = NUM_SC * CH_PER_SC
NUM_PASSES = C // CH_PER_PASS
REGION = CH_PER_SC * PLANE_B
TILE_WB = REGION // NUM_TILES
PER_TILE = HW // NUM_TILES
ZWORDS = TILE_WB // 4


def _decode_body(m_ref, u_ref, d_ref, ut_ref, *, blk):
    m = m_ref[...]
    c = lax.broadcasted_iota(jnp.int32, (blk, C), 1)
    dest = (c % CH_PER_SC) * PLANE_B + m // C
    pad = jnp.zeros((blk, 128 - C), jnp.int32)
    dt = jnp.concatenate([dest, pad], axis=1).T
    ut = jnp.concatenate([u_ref[...], pad.astype(jnp.float32)], axis=1).T
    d_ref[...] = dt[:C, :]
    ut_ref[...] = ut[:C, :]


def _sc_body(dest_hbm, upd_hbm, out_hbm, dest_v, val_v, zero_v, shared, *, b):
    cid = lax.axis_index("c")
    sid = lax.axis_index("s")

    def fill_zero(i, carry):
        zero_v[pl.ds(i * 16, 16)] = jnp.zeros((16,), jnp.float32)
        return carry

    lax.fori_loop(0, ZWORDS // 16, fill_zero, 0)

    def pass_body(p, carry):
        zbase = sid * TILE_WB

        def zcopy(i, c2):
            pltpu.sync_copy(zero_v, shared.at[pl.ds(zbase + i * ZWORDS, ZWORDS)])
            return c2

        lax.fori_loop(0, 4, zcopy, 0)
        plsc.subcore_barrier()

        def ch_body(j, c2):
            ch = p * CH_PER_PASS + cid * CH_PER_SC + j
            col = b * HW + sid * PER_TILE
            pltpu.sync_copy(dest_hbm.at[ch, pl.ds(col, PER_TILE)], dest_v)
            pltpu.sync_copy(upd_hbm.at[ch, pl.ds(col, PER_TILE)], val_v)
            pltpu.sync_copy(val_v, shared.at[dest_v], add=True)
            return c2

        lax.fori_loop(0, CH_PER_SC, ch_body, 0)
        plsc.subcore_barrier()

        row = p * CH_PER_PASS + cid * CH_PER_SC + sid // 2
        col = (sid % 2) * TILE_WB
        pltpu.sync_copy(shared.at[pl.ds(sid * TILE_WB, TILE_WB)],
                        out_hbm.at[row, pl.ds(col, TILE_WB)])
        return carry

    lax.fori_loop(0, NUM_PASSES, pass_body, 0)


def kernel(updates, mask):
    m = mask.astype(jnp.int32).reshape(N_ROWS, C)
    u = updates.reshape(N_ROWS, C)

    blk = 1536
    dest_t, upd_t = pl.pallas_call(
        functools.partial(_decode_body, blk=blk),
        grid=(N_ROWS // blk,),
        in_specs=[
            pl.BlockSpec((blk, C), lambda i: (i, 0)),
            pl.BlockSpec((blk, C), lambda i: (i, 0)),
        ],
        out_specs=[
            pl.BlockSpec((C, blk), lambda i: (0, i)),
            pl.BlockSpec((C, blk), lambda i: (0, i)),
        ],
        out_shape=[
            jax.ShapeDtypeStruct((C, N_ROWS), jnp.int32),
            jax.ShapeDtypeStruct((C, N_ROWS), jnp.float32),
        ],
    )(m, u)

    mesh = plsc.VectorSubcoreMesh(core_axis_name="c", subcore_axis_name="s")
    scratch = [
        pltpu.VMEM((PER_TILE,), jnp.int32),
        pltpu.VMEM((PER_TILE,), jnp.float32),
        pltpu.VMEM((ZWORDS,), jnp.float32),
        pltpu.VMEM_SHARED((REGION,), jnp.float32),
    ]
    parts = []
    for b in range(B):
        sc = pl.kernel(
            functools.partial(_sc_body, b=b),
            out_type=jax.ShapeDtypeStruct((C, PLANE_B), jnp.float32),
            mesh=mesh,
            scratch_types=scratch,
        )
        parts.append(jnp.transpose(sc(dest_t, upd_t)))

    return jnp.stack(parts, axis=0).reshape(B, OH, OW, C)


# speedup vs baseline: 1.1862x; 1.1862x over previous
import functools

import jax
import jax.numpy as jnp
from jax import lax
from jax.experimental import pallas as pl
from jax.experimental.pallas import tpu as pltpu
from jax.experimental.pallas import tpu_sc as plsc

B, H, W, C = 4, 192, 192, 96
OH, OW = 2 * H, 2 * W
N_ROWS = B * H * W
HW = H * W
PLANE_B = OH * OW
OUT_WORDS = PLANE_B * C * B

NUM_SC = 2
NUM_TILES = 16
CH_PER_SC = 8
CH_PER_PASS = NUM_SC * CH_PER_SC
NUM_PASSES = C // CH_PER_PASS
REGION = CH_PER_SC * PLANE_B
TILE_WB = REGION // NUM_TILES
PER_TILE = HW // 2
ZWORDS = TILE_WB // 4


def _decode_body(m_ref, u_ref, d_ref, ut_ref, *, blk):
    m = m_ref[...]
    c = lax.broadcasted_iota(jnp.int32, (blk, C), 1)
    dest = (c % CH_PER_SC) * PLANE_B + m // C
    pad = jnp.zeros((blk, 128 - C), jnp.int32)
    dt = jnp.concatenate([dest, pad], axis=1).T
    ut = jnp.concatenate([u_ref[...], pad.astype(jnp.float32)], axis=1).T
    d_ref[...] = dt[:C, :]
    ut_ref[...] = ut[:C, :]


def _sc_body(dest_hbm, upd_hbm, out_hbm, dest_v, val_v, zero_v, shared, *, b):
    cid = lax.axis_index("c")
    sid = lax.axis_index("s")

    def fill_zero(i, carry):
        zero_v[pl.ds(i * 16, 16)] = jnp.zeros((16,), jnp.float32)
        return carry

    lax.fori_loop(0, ZWORDS // 16, fill_zero, 0)

    def pass_body(p, carry):
        zbase = sid * TILE_WB

        def zcopy(i, c2):
            pltpu.sync_copy(zero_v, shared.at[pl.ds(zbase + i * ZWORDS, ZWORDS)])
            return c2

        lax.fori_loop(0, 4, zcopy, 0)
        plsc.subcore_barrier()

        ch = p * CH_PER_PASS + cid * CH_PER_SC + sid // 2
        col = b * HW + (sid % 2) * PER_TILE
        pltpu.sync_copy(dest_hbm.at[ch, pl.ds(col, PER_TILE)], dest_v)
        pltpu.sync_copy(upd_hbm.at[ch, pl.ds(col, PER_TILE)], val_v)
        pltpu.sync_copy(val_v, shared.at[dest_v], add=True)
        plsc.subcore_barrier()

        row = p * CH_PER_PASS + cid * CH_PER_SC + sid // 2
        col = (sid % 2) * TILE_WB
        pltpu.sync_copy(shared.at[pl.ds(sid * TILE_WB, TILE_WB)],
                        out_hbm.at[row, pl.ds(col, TILE_WB)])
        return carry

    lax.fori_loop(0, NUM_PASSES, pass_body, 0)


def kernel(updates, mask):
    m = mask.astype(jnp.int32).reshape(N_ROWS, C)
    u = updates.reshape(N_ROWS, C)

    blk = 1536
    dest_t, upd_t = pl.pallas_call(
        functools.partial(_decode_body, blk=blk),
        grid=(N_ROWS // blk,),
        in_specs=[
            pl.BlockSpec((blk, C), lambda i: (i, 0)),
            pl.BlockSpec((blk, C), lambda i: (i, 0)),
        ],
        out_specs=[
            pl.BlockSpec((C, blk), lambda i: (0, i)),
            pl.BlockSpec((C, blk), lambda i: (0, i)),
        ],
        out_shape=[
            jax.ShapeDtypeStruct((C, N_ROWS), jnp.int32),
            jax.ShapeDtypeStruct((C, N_ROWS), jnp.float32),
        ],
    )(m, u)

    mesh = plsc.VectorSubcoreMesh(core_axis_name="c", subcore_axis_name="s")
    scratch = [
        pltpu.VMEM((PER_TILE,), jnp.int32),
        pltpu.VMEM((PER_TILE,), jnp.float32),
        pltpu.VMEM((ZWORDS,), jnp.float32),
        pltpu.VMEM_SHARED((REGION,), jnp.float32),
    ]
    parts = []
    for b in range(B):
        sc = pl.kernel(
            functools.partial(_sc_body, b=b),
            out_type=jax.ShapeDtypeStruct((C, PLANE_B), jnp.float32),
            mesh=mesh,
            scratch_types=scratch,
        )
        parts.append(jnp.transpose(sc(dest_t, upd_t)))

    return jnp.stack(parts, axis=0).reshape(B, OH, OW, C)


# speedup vs baseline: 1.2796x; 1.0787x over previous
import functools

import jax
import jax.numpy as jnp
from jax import lax
from jax.experimental import pallas as pl
from jax.experimental.pallas import tpu as pltpu
from jax.experimental.pallas import tpu_sc as plsc

B, H, W, C = 4, 192, 192, 96
OH, OW = 2 * H, 2 * W
N_ROWS = B * H * W
HW = H * W
PLANE_B = OH * OW
OUT_WORDS = PLANE_B * C * B

NUM_SC = 2
NUM_TILES = 16
CH_PER_SC = 8
CH_PER_PASS = NUM_SC * CH_PER_SC
NUM_PASSES = C // CH_PER_PASS
REGION = CH_PER_SC * PLANE_B
TILE_WB = REGION // NUM_TILES
PER_TILE = HW // 2
ZWORDS = TILE_WB // 4


def _decode_body(m_ref, u_ref, d_ref, ut_ref, *, blk):
    m = m_ref[...]
    c = lax.broadcasted_iota(jnp.int32, (blk, C), 1)
    dest = (c % CH_PER_SC) * PLANE_B + m // C
    pad = jnp.zeros((blk, 128 - C), jnp.int32)
    dt = jnp.concatenate([dest, pad], axis=1).T
    ut = jnp.concatenate([u_ref[...], pad.astype(jnp.float32)], axis=1).T
    d_ref[...] = dt[:C, :]
    ut_ref[...] = ut[:C, :]


def _untranspose_body(i_ref, o_ref, *, blk):
    x = i_ref[...]
    pad = jnp.zeros((128 - C, blk), jnp.float32)
    t = jnp.concatenate([x, pad], axis=0).T
    o_ref[...] = t[:, :C]


def _sc_body(dest_hbm, upd_hbm, out_hbm, dest_v, val_v, zero_v, shared, *, b):
    cid = lax.axis_index("c")
    sid = lax.axis_index("s")

    def fill_zero(i, carry):
        zero_v[pl.ds(i * 16, 16)] = jnp.zeros((16,), jnp.float32)
        return carry

    lax.fori_loop(0, ZWORDS // 16, fill_zero, 0)

    def pass_body(p, carry):
        zbase = sid * TILE_WB

        def zcopy(i, c2):
            pltpu.sync_copy(zero_v, shared.at[pl.ds(zbase + i * ZWORDS, ZWORDS)])
            return c2

        lax.fori_loop(0, 4, zcopy, 0)
        plsc.subcore_barrier()

        ch = p * CH_PER_PASS + cid * CH_PER_SC + sid // 2
        col = b * HW + (sid % 2) * PER_TILE
        pltpu.sync_copy(dest_hbm.at[ch, pl.ds(col, PER_TILE)], dest_v)
        pltpu.sync_copy(upd_hbm.at[ch, pl.ds(col, PER_TILE)], val_v)
        pltpu.sync_copy(val_v, shared.at[dest_v], add=True)
        plsc.subcore_barrier()

        row = p * CH_PER_PASS + cid * CH_PER_SC + sid // 2
        col = (sid % 2) * TILE_WB
        pltpu.sync_copy(shared.at[pl.ds(sid * TILE_WB, TILE_WB)],
                        out_hbm.at[row, pl.ds(col, TILE_WB)])
        return carry

    lax.fori_loop(0, NUM_PASSES, pass_body, 0)


def kernel(updates, mask):
    m = mask.astype(jnp.int32).reshape(N_ROWS, C)
    u = updates.reshape(N_ROWS, C)

    blk = 1536
    dest_t, upd_t = pl.pallas_call(
        functools.partial(_decode_body, blk=blk),
        grid=(N_ROWS // blk,),
        in_specs=[
            pl.BlockSpec((blk, C), lambda i: (i, 0)),
            pl.BlockSpec((blk, C), lambda i: (i, 0)),
        ],
        out_specs=[
            pl.BlockSpec((C, blk), lambda i: (0, i)),
            pl.BlockSpec((C, blk), lambda i: (0, i)),
        ],
        out_shape=[
            jax.ShapeDtypeStruct((C, N_ROWS), jnp.int32),
            jax.ShapeDtypeStruct((C, N_ROWS), jnp.float32),
        ],
    )(m, u)

    mesh = plsc.VectorSubcoreMesh(core_axis_name="c", subcore_axis_name="s")
    scratch = [
        pltpu.VMEM((PER_TILE,), jnp.int32),
        pltpu.VMEM((PER_TILE,), jnp.float32),
        pltpu.VMEM((ZWORDS,), jnp.float32),
        pltpu.VMEM_SHARED((REGION,), jnp.float32),
    ]
    blkc = 4608
    untr = pl.pallas_call(
        functools.partial(_untranspose_body, blk=blkc),
        grid=(PLANE_B // blkc,),
        in_specs=[pl.BlockSpec((C, blkc), lambda i: (0, i))],
        out_specs=pl.BlockSpec((blkc, C), lambda i: (i, 0)),
        out_shape=jax.ShapeDtypeStruct((PLANE_B, C), jnp.float32),
    )
    parts = []
    for b in range(B):
        sc = pl.kernel(
            functools.partial(_sc_body, b=b),
            out_type=jax.ShapeDtypeStruct((C, PLANE_B), jnp.float32),
            mesh=mesh,
            scratch_types=scratch,
        )
        parts.append(untr(sc(dest_t, upd_t)))

    return jnp.stack(parts, axis=0).reshape(B, OH, OW, C)


# speedup vs baseline: 1.4083x; 1.1006x over previous
import functools

import jax
import jax.numpy as jnp
from jax import lax
from jax.experimental import pallas as pl
from jax.experimental.pallas import tpu as pltpu
from jax.experimental.pallas import tpu_sc as plsc

B, H, W, C = 4, 192, 192, 96
OH, OW = 2 * H, 2 * W
N_ROWS = B * H * W
HW = H * W
PLANE_B = OH * OW
OUT_WORDS = PLANE_B * C * B

NUM_SC = 2
NUM_TILES = 16
CH_PER_SC = 8
CH_PER_PASS = NUM_SC * CH_PER_SC
NUM_PASSES = C // CH_PER_PASS
REGION = CH_PER_SC * PLANE_B
TILE_WB = REGION // NUM_TILES
PER_TILE = HW // 2
ZWORDS = TILE_WB // 4


def _decode_body(m_ref, u_ref, d_ref, ut_ref, *, blk):
    m = m_ref[...]
    c = lax.broadcasted_iota(jnp.int32, (blk, C), 1)
    dest = (c % CH_PER_SC) * PLANE_B + m // C
    pad = jnp.zeros((blk, 128 - C), jnp.int32)
    dt = jnp.concatenate([dest, pad], axis=1).T
    ut = jnp.concatenate([u_ref[...], pad.astype(jnp.float32)], axis=1).T
    d_ref[...] = dt[:C, :]
    ut_ref[...] = ut[:C, :]


def _untranspose_body(i_ref, o_ref, *, blk):
    x = i_ref[...]
    pad = jnp.zeros((128 - C, blk), jnp.float32)
    t = jnp.concatenate([x, pad], axis=0).T
    o_ref[...] = t[:, :C]


def _sc_body(dest_hbm, upd_hbm, out_hbm, dest_v, val_v, zero_v, shared, *, b):
    cid = lax.axis_index("c")
    sid = lax.axis_index("s")

    def fill_zero(i, carry):
        zero_v[pl.ds(i * 16, 16)] = jnp.zeros((16,), jnp.float32)
        return carry

    lax.fori_loop(0, ZWORDS // 16, fill_zero, 0)

    def pass_body(p, carry):
        zbase = sid * TILE_WB

        def zcopy(i, c2):
            pltpu.sync_copy(zero_v, shared.at[pl.ds(zbase + i * ZWORDS, ZWORDS)])
            return c2

        lax.fori_loop(0, 4, zcopy, 0)
        plsc.subcore_barrier()

        ch = p * CH_PER_PASS + cid * CH_PER_SC + sid // 2
        col = (sid % 2) * PER_TILE
        pltpu.sync_copy(dest_hbm.at[ch, pl.ds(col, PER_TILE)], dest_v)
        pltpu.sync_copy(upd_hbm.at[ch, pl.ds(col, PER_TILE)], val_v)
        pltpu.sync_copy(val_v, shared.at[dest_v], add=True)
        plsc.subcore_barrier()

        row = p * CH_PER_PASS + cid * CH_PER_SC + sid // 2
        col = (sid % 2) * TILE_WB
        pltpu.sync_copy(shared.at[pl.ds(sid * TILE_WB, TILE_WB)],
                        out_hbm.at[row, pl.ds(col, TILE_WB)])
        return carry

    lax.fori_loop(0, NUM_PASSES, pass_body, 0)


def kernel(updates, mask):
    m = mask.astype(jnp.int32).reshape(N_ROWS, C)
    u = updates.reshape(N_ROWS, C)

    blk = 1536
    rb = HW // blk

    def _decode_batch(b):
        return pl.pallas_call(
            functools.partial(_decode_body, blk=blk),
            grid=(rb,),
            in_specs=[
                pl.BlockSpec((blk, C), lambda i, b=b: (b * rb + i, 0)),
                pl.BlockSpec((blk, C), lambda i, b=b: (b * rb + i, 0)),
            ],
            out_specs=[
                pl.BlockSpec((C, blk), lambda i: (0, i)),
                pl.BlockSpec((C, blk), lambda i: (0, i)),
            ],
            out_shape=[
                jax.ShapeDtypeStruct((C, HW), jnp.int32),
                jax.ShapeDtypeStruct((C, HW), jnp.float32),
            ],
        )(m, u)

    mesh = plsc.VectorSubcoreMesh(core_axis_name="c", subcore_axis_name="s")
    scratch = [
        pltpu.VMEM((PER_TILE,), jnp.int32),
        pltpu.VMEM((PER_TILE,), jnp.float32),
        pltpu.VMEM((ZWORDS,), jnp.float32),
        pltpu.VMEM_SHARED((REGION,), jnp.float32),
    ]
    blkc = 4608
    untr = pl.pallas_call(
        functools.partial(_untranspose_body, blk=blkc),
        grid=(PLANE_B // blkc,),
        in_specs=[pl.BlockSpec((C, blkc), lambda i: (0, i))],
        out_specs=pl.BlockSpec((blkc, C), lambda i: (i, 0)),
        out_shape=jax.ShapeDtypeStruct((PLANE_B, C), jnp.float32),
    )
    parts = []
    for b in range(B):
        dest_tb, upd_tb = _decode_batch(b)
        sc = pl.kernel(
            functools.partial(_sc_body, b=b),
            out_type=jax.ShapeDtypeStruct((C, PLANE_B), jnp.float32),
            mesh=mesh,
            scratch_types=scratch,
        )
        parts.append(untr(sc(dest_tb, upd_tb)))

    return jnp.stack(parts, axis=0).reshape(B, OH, OW, C)
